# baseline (speedup 1.0000x reference)
import jax
import jax.numpy as jnp
from jax import lax
from jax.experimental import pallas as pl
from jax.experimental.pallas import tpu as pltpu
from jax.experimental.pallas import tpu_sc as plsc

VOCAB = 256
D_MODEL = 384

_NC = 2
_NS = 16
_NW = _NC * _NS

_N = 4 * 8192
_BPW = _N // _NW
_CH = 32
_NCH = _BPW // _CH
_NBUF = 2
_L = 16
_COLS = D_MODEL // _L


def _table_body(e_ref, w_ref, b_ref, t_ref):
    t_ref[...] = lax.dot_general(
        e_ref[...], w_ref[...],
        dimension_numbers=(((1,), (1,)), ((), ())),
        preferred_element_type=jnp.float32,
    ) + b_ref[...]


_fuse_table = pl.pallas_call(
    _table_body,
    out_shape=jax.ShapeDtypeStruct((VOCAB, D_MODEL), jnp.float32),
)


def _gather_body(ids_hbm, table_hbm, out_hbm, idx_v, rows_v, table_v,
                 table_sp, s0, s1):
    sid = lax.axis_index("s")
    wid = sid * _NC + lax.axis_index("c")
    base = wid * _BPW

    del table_sp, sid
    pltpu.sync_copy(ids_hbm.at[pl.ds(base, _BPW)], idx_v)
    pltpu.sync_copy(table_hbm, table_v)

    ssems = (s0, s1)
    iota = lax.iota(jnp.int32, _L)

    def compute_chunk(c, b):
        for gg in range(_CH // _L):
            idvec = idx_v[pl.ds(c * _CH + gg * _L, _L)]
            idvec = jnp.clip(idvec, 0, VOCAB - 1) * D_MODEL
            for l in range(_L):
                lane = jnp.full((_L,), l, jnp.int32)
                addr0 = jnp.take_along_axis(
                    idvec, lane, axis=0,
                    mode=lax.GatherScatterMode.PROMISE_IN_BOUNDS) + iota
                row = gg * _L + l
                vals = [plsc.load_gather(table_v, [addr0 + (jj * _L)])
                        for jj in range(_COLS)]
                for jj in range(_COLS):
                    rows_v[b, row, pl.ds(jj * _L, _L)] = vals[jj]

    def outer(o, _):
        for b in range(_NBUF):
            c = o * _NBUF + b

            @pl.when(o > 0)
            def _():
                pltpu.make_async_copy(
                    rows_v.at[b], out_hbm.at[pl.ds(base, _CH)],
                    ssems[b]).wait()

            compute_chunk(c, b)
            pltpu.async_copy(
                rows_v.at[b],
                out_hbm.at[pl.ds(base + c * _CH, _CH)], ssems[b])
        return 0

    lax.fori_loop(0, _NCH // _NBUF, outer, 0)
    for b in range(_NBUF):
        pltpu.make_async_copy(
            rows_v.at[b], out_hbm.at[pl.ds(base, _CH)], ssems[b]).wait()


_gather = pl.kernel(
    _gather_body,
    out_type=jax.ShapeDtypeStruct((_N, D_MODEL), jnp.float32),
    mesh=plsc.VectorSubcoreMesh(core_axis_name="c", subcore_axis_name="s"),
    compiler_params=pltpu.CompilerParams(needs_layout_passes=False),
    scratch_types=[
        pltpu.VMEM((_BPW,), jnp.int32),
        pltpu.VMEM((_NBUF, _CH, D_MODEL), jnp.float32),
        pltpu.VMEM((VOCAB * D_MODEL,), jnp.float32),
        pltpu.VMEM_SHARED((VOCAB * D_MODEL,), jnp.float32),
        pltpu.SemaphoreType.DMA,
        pltpu.SemaphoreType.DMA,
    ],
)


@jax.jit
def kernel(byte_ids, embed_table, W, b):
    table = _fuse_table(embed_table, W, b.reshape(1, D_MODEL))
    ids = byte_ids.reshape(-1)
    out = _gather(ids, table.reshape(-1))
    return out.reshape(byte_ids.shape + (D_MODEL,))


# speedup vs baseline: 1.4249x; 1.4249x over previous
import jax
import jax.numpy as jnp
from jax import lax
from jax.experimental import pallas as pl
from jax.experimental.pallas import tpu as pltpu
from jax.experimental.pallas import tpu_sc as plsc

VOCAB = 256
D_MODEL = 384

_NC = 2
_NS = 16
_NW = _NC * _NS

_N = 4 * 8192
_BPW = _N // _NW
_CH = 128
_NCH = _BPW // _CH
_NBUF = 2


def _table_body(e_ref, w_ref, b_ref, t_ref):
    t_ref[...] = lax.dot_general(
        e_ref[...], w_ref[...],
        dimension_numbers=(((1,), (1,)), ((), ())),
        preferred_element_type=jnp.float32,
    ) + b_ref[...]


_fuse_table = pl.pallas_call(
    _table_body,
    out_shape=jax.ShapeDtypeStruct((VOCAB, D_MODEL), jnp.float32),
)


def _gather_body(ids_hbm, table_hbm, out_hbm, idx_v, rows_v, *sems):
    wid = lax.axis_index("s") * _NC + lax.axis_index("c")
    base = wid * _BPW

    pltpu.sync_copy(ids_hbm.at[pl.ds(base, _BPW)], idx_v)
    for i in range(_BPW // 16):
        sl = pl.ds(i * 16, 16)
        idx_v[sl] = jnp.clip(idx_v[sl], 0, VOCAB - 1)

    gsems = sems[:_NBUF]
    ssems = sems[_NBUF:]
    gh = [None] * _NBUF
    sh = [None] * _NBUF

    def start_gather(c):
        buf = c % _NBUF
        if sh[buf] is not None:
            sh[buf].wait()
        gh[buf] = pltpu.async_copy(
            table_hbm.at[idx_v.at[pl.ds(c * _CH, _CH)]],
            rows_v.at[buf], gsems[buf])

    for k in range(min(_NBUF - 1, _NCH)):
        start_gather(k)
    for c in range(_NCH):
        buf = c % _NBUF
        gh[buf].wait()
        sh[buf] = pltpu.async_copy(
            rows_v.at[buf],
            out_hbm.at[pl.ds(base + c * _CH, _CH)], ssems[buf])
        nxt = c + _NBUF - 1
        if nxt < _NCH:
            start_gather(nxt)
    for buf in range(_NBUF):
        if sh[buf] is not None:
            sh[buf].wait()


_gather = pl.kernel(
    _gather_body,
    out_type=jax.ShapeDtypeStruct((_N, D_MODEL), jnp.float32),
    mesh=plsc.VectorSubcoreMesh(core_axis_name="c", subcore_axis_name="s"),
    scratch_types=[
        pltpu.VMEM((_BPW,), jnp.int32),
        pltpu.VMEM((_NBUF, _CH, D_MODEL), jnp.float32),
    ] + [pltpu.SemaphoreType.DMA] * (2 * _NBUF),
)


@jax.jit
def kernel(byte_ids, embed_table, W, b):
    table = _fuse_table(embed_table, W, b.reshape(1, D_MODEL))
    ids = byte_ids.reshape(-1)
    out = _gather(ids, table)
    return out.reshape(byte_ids.shape + (D_MODEL,))
